# Initial kernel scaffold; baseline (speedup 1.0000x reference)
import functools

import jax
import jax.numpy as jnp
from jax import lax
from jax.experimental import pallas as pl
from jax.experimental.pallas import tpu as pltpu
from jax.experimental.pallas import tpu_sc as plsc

N = 10000
NPAD = 10240
E = 320000
D = 128

NC, NS = 2, 16
NW = NC * NS
B = 128
NBLK = -(-E // (B * NW))
EPAD = NBLK * B * NW
ROWS_PER_SUB = NPAD // NS

_MESH = plsc.VectorSubcoreMesh(core_axis_name="c", subcore_axis_name="s")


def _zero_tile(zb, width):
    zero = jnp.zeros((16,), jnp.float32)

    @pl.loop(0, 128)
    def _(r):
        @pl.loop(0, width, step=16)
        def _(col):
            zb[r, pl.ds(col, 16)] = zero


def _zero_shared_rows(zb, acc, s):
    @pl.loop(0, ROWS_PER_SUB, step=128)
    def _(r0):
        pltpu.sync_copy(zb, acc.at[pl.ds(s * ROWS_PER_SUB + r0, 128)])


@jax.jit
def _sc_degree(dstp):
    @functools.partial(
        pl.kernel,
        mesh=_MESH,
        out_type=jax.ShapeDtypeStruct((NC, NPAD, 16), jnp.float32),
        scratch_types=[
            pltpu.VMEM((B,), jnp.int32),
            pltpu.VMEM((B, 16), jnp.float32),
            pltpu.VMEM((128, 16), jnp.float32),
            pltpu.VMEM_SHARED((NPAD, 16), jnp.float32),
        ],
    )
    def k(dst_hbm, out_hbm, didx, ones_v, zb, acc):
        c = lax.axis_index("c")
        s = lax.axis_index("s")
        wid = s * NC + c

        one = jnp.full((16,), 1.0, jnp.float32)

        @pl.loop(0, B)
        def _(r):
            ones_v[r] = one

        _zero_tile(zb, 16)
        _zero_shared_rows(zb, acc, s)
        plsc.subcore_barrier()

        @pl.loop(0, NBLK)
        def _(i):
            off = (wid * NBLK + i) * B
            pltpu.sync_copy(dst_hbm.at[pl.ds(off, B)], didx)
            pltpu.sync_copy(ones_v, acc.at[didx], add=True)

        plsc.subcore_barrier()
        row0 = s * ROWS_PER_SUB
        pltpu.sync_copy(
            acc.at[pl.ds(row0, ROWS_PER_SUB)],
            out_hbm.at[c].at[pl.ds(row0, ROWS_PER_SUB)],
        )

    return k(dstp)


@jax.jit
def _sc_edge_pass(table, srcp, dstp):
    @functools.partial(
        pl.kernel,
        mesh=_MESH,
        out_type=jax.ShapeDtypeStruct((NC, NPAD, D), jnp.float32),
        scratch_types=[
            pltpu.VMEM((B,), jnp.int32),
            pltpu.VMEM((B,), jnp.int32),
            pltpu.VMEM((B, D), jnp.float32),
            pltpu.VMEM((128, D), jnp.float32),
            pltpu.VMEM_SHARED((NPAD, D), jnp.float32),
            pltpu.SemaphoreType.DMA,
        ],
    )
    def k(tab_hbm, src_hbm, dst_hbm, out_hbm, sidx, didx, rows, zb, acc, sem):
        c = lax.axis_index("c")
        s = lax.axis_index("s")
        wid = s * NC + c

        _zero_tile(zb, D)
        _zero_shared_rows(zb, acc, s)
        plsc.subcore_barrier()

        @pl.loop(0, NBLK)
        def _(i):
            off = (wid * NBLK + i) * B
            pltpu.sync_copy(src_hbm.at[pl.ds(off, B)], sidx)
            pltpu.sync_copy(dst_hbm.at[pl.ds(off, B)], didx)
            pltpu.async_copy(tab_hbm.at[sidx], rows, sem).wait()
            pltpu.sync_copy(rows, acc.at[didx], add=True)

        plsc.subcore_barrier()
        row0 = s * ROWS_PER_SUB
        pltpu.sync_copy(
            acc.at[pl.ds(row0, ROWS_PER_SUB)],
            out_hbm.at[c].at[pl.ds(row0, ROWS_PER_SUB)],
        )

    return k(table, srcp, dstp)


def _dinv(deg_block):
    deg = deg_block[0, :, 0:1] + deg_block[1, :, 0:1]
    return jnp.where(deg > 0, lax.rsqrt(jnp.maximum(deg, 1e-12)), 0.0)


def _mm1_body(x_ref, w_ref, d_ref, o_ref):
    h = jnp.dot(x_ref[...], w_ref[...], preferred_element_type=jnp.float32,
                precision=lax.Precision.HIGHEST)
    o_ref[...] = h * _dinv(d_ref[...])


def _mid_body(p_ref, d_ref, b_ref, w_ref, o_ref):
    dinv = _dinv(d_ref[...])
    y = jnp.maximum((p_ref[0] + p_ref[1]) * dinv + b_ref[...], 0.0)
    o_ref[...] = jnp.dot(y, w_ref[...], preferred_element_type=jnp.float32,
                         precision=lax.Precision.HIGHEST) * dinv


def _final_body(q_ref, d_ref, b_ref, o_ref):
    o_ref[...] = (q_ref[0] + q_ref[1]) * _dinv(d_ref[...]) + b_ref[...]


@jax.jit
def _tc_mm1(Xp, W1, degp):
    R, G = 512, NPAD // 512
    return pl.pallas_call(
        _mm1_body,
        grid=(G,),
        in_specs=[
            pl.BlockSpec((R, D), lambda i: (i, 0)),
            pl.BlockSpec((D, D), lambda i: (0, 0)),
            pl.BlockSpec((NC, R, 16), lambda i: (0, i, 0)),
        ],
        out_specs=pl.BlockSpec((R, D), lambda i: (i, 0)),
        out_shape=jax.ShapeDtypeStruct((NPAD, D), jnp.float32),
    )(Xp, W1, degp)


@jax.jit
def _tc_mid(p, degp, b1, W2):
    R, G = 512, NPAD // 512
    return pl.pallas_call(
        _mid_body,
        grid=(G,),
        in_specs=[
            pl.BlockSpec((NC, R, D), lambda i: (0, i, 0)),
            pl.BlockSpec((NC, R, 16), lambda i: (0, i, 0)),
            pl.BlockSpec((1, D), lambda i: (0, 0)),
            pl.BlockSpec((D, D), lambda i: (0, 0)),
        ],
        out_specs=pl.BlockSpec((R, D), lambda i: (i, 0)),
        out_shape=jax.ShapeDtypeStruct((NPAD, D), jnp.float32),
    )(p, degp, b1.reshape(1, D), W2)


@jax.jit
def _tc_final(q, degp, b2):
    R, G = 2000, 5
    return pl.pallas_call(
        _final_body,
        grid=(G,),
        in_specs=[
            pl.BlockSpec((NC, R, D), lambda i: (0, i, 0)),
            pl.BlockSpec((NC, R, 16), lambda i: (0, i, 0)),
            pl.BlockSpec((1, D), lambda i: (0, 0)),
        ],
        out_specs=pl.BlockSpec((R, D), lambda i: (i, 0)),
        out_shape=jax.ShapeDtypeStruct((N, D), jnp.float32),
    )(q, degp, b2.reshape(1, D))


@jax.jit
def kernel(X, A, W1, b1, W2, b2):
    pad = jnp.asarray(jnp.arange(EPAD - E) % 16 + N, jnp.int32)
    srcp = jnp.concatenate([A[0].astype(jnp.int32), pad])
    dstp = jnp.concatenate([A[1].astype(jnp.int32), pad])
    Xp = jnp.zeros((NPAD, D), jnp.float32).at[:N].set(X)

    degp = _sc_degree(dstp)
    hs = _tc_mm1(Xp, W1, degp)
    p = _sc_edge_pass(hs, srcp, dstp)
    hs2 = _tc_mid(p, degp, b1, W2)
    q = _sc_edge_pass(hs2, srcp, dstp)
    return _tc_final(q, degp, b2)


# speedup vs baseline: 12.4127x; 12.4127x over previous
import functools

import jax
import jax.numpy as jnp
from jax import lax
from jax.experimental import pallas as pl
from jax.experimental.pallas import tpu as pltpu
from jax.experimental.pallas import tpu_sc as plsc

N = 10000
NPAD = 10240
E = 320000
D = 128

NC, NS = 2, 16
NW = NC * NS
B = 128
NBLK = -(-E // (B * NW))
EPAD = NBLK * B * NW
ROWS_PER_SUB = NPAD // NS

_MESH = plsc.VectorSubcoreMesh(core_axis_name="c", subcore_axis_name="s")


def _zero_tile(zb, width):
    zero = jnp.zeros((16,), jnp.float32)

    @pl.loop(0, 128)
    def _(r):
        @pl.loop(0, width, step=16)
        def _(col):
            zb[r, pl.ds(col, 16)] = zero


def _zero_shared_rows(zb, acc, s):
    @pl.loop(0, ROWS_PER_SUB, step=128)
    def _(r0):
        pltpu.sync_copy(zb, acc.at[pl.ds(s * ROWS_PER_SUB + r0, 128)])


@jax.jit
def _sc_degree(dstp):
    @functools.partial(
        pl.kernel,
        mesh=_MESH,
        out_type=jax.ShapeDtypeStruct((NC, NPAD, D), jnp.float32),
        scratch_types=[
            pltpu.VMEM((B,), jnp.int32),
            pltpu.VMEM((B, D), jnp.float32),
            pltpu.VMEM((128, D), jnp.float32),
            pltpu.VMEM_SHARED((NPAD, D), jnp.float32),
        ],
    )
    def k(dst_hbm, out_hbm, didx, ones_v, zb, acc):
        c = lax.axis_index("c")
        s = lax.axis_index("s")
        wid = s * NC + c

        one = jnp.full((16,), 1.0, jnp.float32)

        @pl.loop(0, B)
        def _(r):
            @pl.loop(0, D, step=16)
            def _(col):
                ones_v[r, pl.ds(col, 16)] = one

        _zero_tile(zb, D)
        _zero_shared_rows(zb, acc, s)
        plsc.subcore_barrier()

        @pl.loop(0, NBLK)
        def _(i):
            off = (wid * NBLK + i) * B
            pltpu.sync_copy(dst_hbm.at[pl.ds(off, B)], didx)
            pltpu.sync_copy(ones_v, acc.at[didx], add=True)

        plsc.subcore_barrier()
        row0 = s * ROWS_PER_SUB
        pltpu.sync_copy(
            acc.at[pl.ds(row0, ROWS_PER_SUB)],
            out_hbm.at[c].at[pl.ds(row0, ROWS_PER_SUB)],
        )

    return k(dstp)


@jax.jit
def _sc_edge_pass(table, srcp, dstp):
    @functools.partial(
        pl.kernel,
        mesh=_MESH,
        out_type=jax.ShapeDtypeStruct((NC, NPAD, D), jnp.float32),
        scratch_types=[
            pltpu.VMEM((B,), jnp.int32),
            pltpu.VMEM((B,), jnp.int32),
            pltpu.VMEM((B, D), jnp.float32),
            pltpu.VMEM((128, D), jnp.float32),
            pltpu.VMEM_SHARED((NPAD, D), jnp.float32),
            pltpu.SemaphoreType.DMA,
        ],
    )
    def k(tab_hbm, src_hbm, dst_hbm, out_hbm, sidx, didx, rows, zb, acc, sem):
        c = lax.axis_index("c")
        s = lax.axis_index("s")
        wid = s * NC + c

        _zero_tile(zb, D)
        _zero_shared_rows(zb, acc, s)
        plsc.subcore_barrier()

        @pl.loop(0, NBLK)
        def _(i):
            off = (wid * NBLK + i) * B
            pltpu.sync_copy(src_hbm.at[pl.ds(off, B)], sidx)
            pltpu.sync_copy(dst_hbm.at[pl.ds(off, B)], didx)
            pltpu.async_copy(tab_hbm.at[sidx], rows, sem).wait()
            pltpu.sync_copy(rows, acc.at[didx], add=True)

        plsc.subcore_barrier()
        row0 = s * ROWS_PER_SUB
        pltpu.sync_copy(
            acc.at[pl.ds(row0, ROWS_PER_SUB)],
            out_hbm.at[c].at[pl.ds(row0, ROWS_PER_SUB)],
        )

    return k(table, srcp, dstp)


def _dinv(deg_block):
    deg = deg_block[0, :, 0:1] + deg_block[1, :, 0:1]
    return jnp.where(deg > 0, lax.rsqrt(jnp.maximum(deg, 1e-12)), 0.0)


def _mm1_body(x_ref, w_ref, d_ref, o_ref):
    h = jnp.dot(x_ref[...], w_ref[...], preferred_element_type=jnp.float32,
                precision=lax.Precision.HIGHEST)
    o_ref[...] = h * _dinv(d_ref[...])


def _mid_body(p_ref, d_ref, b_ref, w_ref, o_ref):
    dinv = _dinv(d_ref[...])
    y = jnp.maximum((p_ref[0] + p_ref[1]) * dinv + b_ref[...], 0.0)
    o_ref[...] = jnp.dot(y, w_ref[...], preferred_element_type=jnp.float32,
                         precision=lax.Precision.HIGHEST) * dinv


def _final_body(q_ref, d_ref, b_ref, o_ref):
    o_ref[...] = (q_ref[0] + q_ref[1]) * _dinv(d_ref[...]) + b_ref[...]


@jax.jit
def _tc_mm1(Xp, W1, degp):
    R, G = 512, NPAD // 512
    return pl.pallas_call(
        _mm1_body,
        grid=(G,),
        in_specs=[
            pl.BlockSpec((R, D), lambda i: (i, 0)),
            pl.BlockSpec((D, D), lambda i: (0, 0)),
            pl.BlockSpec((NC, R, D), lambda i: (0, i, 0)),
        ],
        out_specs=pl.BlockSpec((R, D), lambda i: (i, 0)),
        out_shape=jax.ShapeDtypeStruct((NPAD, D), jnp.float32),
    )(Xp, W1, degp)


@jax.jit
def _tc_mid(p, degp, b1, W2):
    R, G = 512, NPAD // 512
    return pl.pallas_call(
        _mid_body,
        grid=(G,),
        in_specs=[
            pl.BlockSpec((NC, R, D), lambda i: (0, i, 0)),
            pl.BlockSpec((NC, R, D), lambda i: (0, i, 0)),
            pl.BlockSpec((1, D), lambda i: (0, 0)),
            pl.BlockSpec((D, D), lambda i: (0, 0)),
        ],
        out_specs=pl.BlockSpec((R, D), lambda i: (i, 0)),
        out_shape=jax.ShapeDtypeStruct((NPAD, D), jnp.float32),
    )(p, degp, b1.reshape(1, D), W2)


@jax.jit
def _tc_final(q, degp, b2):
    R, G = 2000, 5
    return pl.pallas_call(
        _final_body,
        grid=(G,),
        in_specs=[
            pl.BlockSpec((NC, R, D), lambda i: (0, i, 0)),
            pl.BlockSpec((NC, R, D), lambda i: (0, i, 0)),
            pl.BlockSpec((1, D), lambda i: (0, 0)),
        ],
        out_specs=pl.BlockSpec((R, D), lambda i: (i, 0)),
        out_shape=jax.ShapeDtypeStruct((N, D), jnp.float32),
    )(q, degp, b2.reshape(1, D))


@jax.jit
def kernel(X, A, W1, b1, W2, b2):
    pad = jnp.asarray(jnp.arange(EPAD - E) % 16 + N, jnp.int32)
    srcp = jnp.concatenate([A[0].astype(jnp.int32), pad])
    dstp = jnp.concatenate([A[1].astype(jnp.int32), pad])
    Xp = jnp.zeros((NPAD, D), jnp.float32).at[:N].set(X)

    degp = _sc_degree(dstp)
    hs = _tc_mm1(Xp, W1, degp)
    p = _sc_edge_pass(hs, srcp, dstp)
    hs2 = _tc_mid(p, degp, b1, W2)
    q = _sc_edge_pass(hs2, srcp, dstp)
    return _tc_final(q, degp, b2)


# speedup vs baseline: 16.3230x; 1.3150x over previous
import functools

import jax
import jax.numpy as jnp
from jax import lax
from jax.experimental import pallas as pl
from jax.experimental.pallas import tpu as pltpu
from jax.experimental.pallas import tpu_sc as plsc

N = 10000
NPAD = 10112
E = 320000
D = 128

NC, NS = 2, 16
NW = NC * NS
B = 128
DEPTH = 3
NBLK = 81
EPAD = NBLK * B * NW
ROWS_PER_SUB = NPAD // NS

_MESH = plsc.VectorSubcoreMesh(core_axis_name="c", subcore_axis_name="s")


def _zero_tile(zb, width):
    zero = jnp.zeros((16,), jnp.float32)

    @pl.loop(0, 128)
    def _(r):
        @pl.loop(0, width, step=16)
        def _(col):
            zb[r, pl.ds(col, 16)] = zero


def _zero_shared_rows(zb, acc, s):
    tail = ROWS_PER_SUB % 128

    @pl.loop(0, ROWS_PER_SUB - tail, step=128)
    def _(r0):
        pltpu.sync_copy(zb, acc.at[pl.ds(s * ROWS_PER_SUB + r0, 128)])

    if tail:
        pltpu.sync_copy(
            zb.at[pl.ds(0, tail)],
            acc.at[pl.ds(s * ROWS_PER_SUB + ROWS_PER_SUB - tail, tail)],
        )


@jax.jit
def _sc_degree(sd):
    @functools.partial(
        pl.kernel,
        mesh=_MESH,
        out_type=jax.ShapeDtypeStruct((NC, NPAD, D), jnp.float32),
        scratch_types=(
            [pltpu.VMEM((2, B), jnp.int32) for _ in range(DEPTH)]
            + [
                pltpu.VMEM((B, D), jnp.float32),
                pltpu.VMEM((128, D), jnp.float32),
                pltpu.VMEM_SHARED((NPAD, D), jnp.float32),
                pltpu.SemaphoreType.DMA,
            ]
        ),
    )
    def k(sd_hbm, out_hbm, idx0, idx1, idx2, ones_v, zb, acc, sem):
        idx = [idx0, idx1, idx2]
        c = lax.axis_index("c")
        s = lax.axis_index("s")
        wid = s * NC + c
        row0 = s * ROWS_PER_SUB
        blk0 = wid * NBLK

        one = jnp.full((16,), 1.0, jnp.float32)

        @pl.loop(0, B)
        def _(r):
            @pl.loop(0, D, step=16)
            def _(col):
                ones_v[r, pl.ds(col, 16)] = one

        _zero_tile(zb, D)
        _zero_shared_rows(zb, acc, s)
        plsc.subcore_barrier()

        @pl.loop(0, NBLK, step=DEPTH)
        def _(i):
            idx_cps = [
                pltpu.async_copy(sd_hbm.at[blk0 + i + j], idx[j], sem)
                for j in range(DEPTH)
            ]
            for cp in idx_cps:
                cp.wait()
            for j in range(DEPTH):
                pltpu.sync_copy(ones_v, acc.at[idx[j].at[1]], add=True)

        plsc.subcore_barrier()
        pltpu.sync_copy(
            acc.at[pl.ds(row0, ROWS_PER_SUB)],
            out_hbm.at[c].at[pl.ds(row0, ROWS_PER_SUB)],
        )

    return k(sd)


@jax.jit
def _sc_edge_pass(table, sd):
    @functools.partial(
        pl.kernel,
        mesh=_MESH,
        out_type=jax.ShapeDtypeStruct((NC, NPAD, D), jnp.float32),
        scratch_types=(
            [pltpu.VMEM((2, B), jnp.int32) for _ in range(DEPTH)]
            + [pltpu.VMEM((B, D), jnp.float32) for _ in range(DEPTH)]
            + [
                pltpu.VMEM_SHARED((NPAD, D), jnp.float32),
                pltpu.SemaphoreType.DMA,
            ]
            + [pltpu.SemaphoreType.DMA for _ in range(DEPTH)]
        ),
    )
    def k(tab_hbm, sd_hbm, out_hbm,
          idx0, idx1, idx2, rows0, rows1, rows2, acc,
          sem_i, sem_g0, sem_g1, sem_g2):
        c = lax.axis_index("c")
        s = lax.axis_index("s")
        wid = s * NC + c
        blk0 = wid * NBLK
        idx = [idx0, idx1, idx2]
        rows = [rows0, rows1, rows2]
        sem_g = [sem_g0, sem_g1, sem_g2]

        _zero_tile(rows0, D)
        _zero_shared_rows(rows0, acc, s)
        plsc.subcore_barrier()

        @pl.loop(0, NBLK, step=DEPTH)
        def _(i):
            idx_cps = [
                pltpu.async_copy(sd_hbm.at[blk0 + i + j], idx[j], sem_i)
                for j in range(DEPTH)
            ]
            for cp in idx_cps:
                cp.wait()
            g_cps = [
                pltpu.async_copy(tab_hbm.at[idx[j].at[0]], rows[j], sem_g[j])
                for j in range(DEPTH)
            ]
            for j in range(DEPTH):
                g_cps[j].wait()
                pltpu.sync_copy(rows[j], acc.at[idx[j].at[1]], add=True)

        plsc.subcore_barrier()
        row0 = s * ROWS_PER_SUB
        pltpu.sync_copy(
            acc.at[pl.ds(row0, ROWS_PER_SUB)],
            out_hbm.at[c].at[pl.ds(row0, ROWS_PER_SUB)],
        )

    return k(table, sd)


def _dinv(deg_block):
    deg = deg_block[0, :, 0:1] + deg_block[1, :, 0:1]
    return jnp.where(deg > 0, lax.rsqrt(jnp.maximum(deg, 1e-12)), 0.0)


def _mm1_body(x_ref, w_ref, d_ref, o_ref):
    h = jnp.dot(x_ref[...], w_ref[...], preferred_element_type=jnp.float32,
                precision=lax.Precision.HIGHEST)
    o_ref[...] = h * _dinv(d_ref[...])


def _mid_body(p_ref, d_ref, b_ref, w_ref, o_ref):
    dinv = _dinv(d_ref[...])
    y = jnp.maximum((p_ref[0] + p_ref[1]) * dinv + b_ref[...], 0.0)
    o_ref[...] = jnp.dot(y, w_ref[...], preferred_element_type=jnp.float32,
                         precision=lax.Precision.HIGHEST) * dinv


def _final_body(q_ref, d_ref, b_ref, o_ref):
    o_ref[...] = (q_ref[0] + q_ref[1]) * _dinv(d_ref[...]) + b_ref[...]


@jax.jit
def _tc_mm1(Xp, W1, degp):
    R, G = 632, 16
    return pl.pallas_call(
        _mm1_body,
        grid=(G,),
        in_specs=[
            pl.BlockSpec((R, D), lambda i: (i, 0)),
            pl.BlockSpec((D, D), lambda i: (0, 0)),
            pl.BlockSpec((NC, R, D), lambda i: (0, i, 0)),
        ],
        out_specs=pl.BlockSpec((R, D), lambda i: (i, 0)),
        out_shape=jax.ShapeDtypeStruct((NPAD, D), jnp.float32),
    )(Xp, W1, degp)


@jax.jit
def _tc_mid(p, degp, b1, W2):
    R, G = 632, 16
    return pl.pallas_call(
        _mid_body,
        grid=(G,),
        in_specs=[
            pl.BlockSpec((NC, R, D), lambda i: (0, i, 0)),
            pl.BlockSpec((NC, R, D), lambda i: (0, i, 0)),
            pl.BlockSpec((1, D), lambda i: (0, 0)),
            pl.BlockSpec((D, D), lambda i: (0, 0)),
        ],
        out_specs=pl.BlockSpec((R, D), lambda i: (i, 0)),
        out_shape=jax.ShapeDtypeStruct((NPAD, D), jnp.float32),
    )(p, degp, b1.reshape(1, D), W2)


@jax.jit
def _tc_final(q, degp, b2):
    R, G = 2000, 5
    return pl.pallas_call(
        _final_body,
        grid=(G,),
        in_specs=[
            pl.BlockSpec((NC, R, D), lambda i: (0, i, 0)),
            pl.BlockSpec((NC, R, D), lambda i: (0, i, 0)),
            pl.BlockSpec((1, D), lambda i: (0, 0)),
        ],
        out_specs=pl.BlockSpec((R, D), lambda i: (i, 0)),
        out_shape=jax.ShapeDtypeStruct((N, D), jnp.float32),
    )(q, degp, b2.reshape(1, D))


@jax.jit
def kernel(X, A, W1, b1, W2, b2):
    pad = jnp.asarray(jnp.arange(EPAD - E) % 16 + N, jnp.int32)
    srcp = jnp.concatenate([A[0].astype(jnp.int32), pad])
    dstp = jnp.concatenate([A[1].astype(jnp.int32), pad])
    sd = jnp.stack([srcp.reshape(EPAD // B, B), dstp.reshape(EPAD // B, B)],
                   axis=1)
    Xp = jnp.zeros((NPAD, D), jnp.float32).at[:N].set(X)

    degp = _sc_degree(sd)
    hs = _tc_mm1(Xp, W1, degp)
    p = _sc_edge_pass(hs, sd)
    hs2 = _tc_mid(p, degp, b1, W2)
    q = _sc_edge_pass(hs2, sd)
    return _tc_final(q, degp, b2)


# speedup vs baseline: 21.2760x; 1.3034x over previous
import functools

import jax
import jax.numpy as jnp
from jax import lax
from jax.experimental import pallas as pl
from jax.experimental.pallas import tpu as pltpu
from jax.experimental.pallas import tpu_sc as plsc

N = 10000
NPAD = 10016
E = 320000
D = 128

NC, NS = 2, 16
NW = NC * NS
B = 128
DEPTH = 3
NBLK = 81
EPAD = NBLK * B * NW
SUB_ROWS = 632
SUB_ROWS_LAST = NPAD - 15 * SUB_ROWS

_MESH = plsc.VectorSubcoreMesh(core_axis_name="c", subcore_axis_name="s")


def _zero_tile(zb, width):
    zero = jnp.zeros((16,), jnp.float32)

    @pl.loop(0, 128)
    def _(r):
        @pl.loop(0, width, step=16)
        def _(col):
            zb[r, pl.ds(col, 16)] = zero


def _zero_rows(zb, acc, row0, nrows):
    tail = nrows % 128

    @pl.loop(0, nrows - tail, step=128)
    def _(r0):
        pltpu.sync_copy(zb, acc.at[pl.ds(row0 + r0, 128)])

    if tail:
        pltpu.sync_copy(
            zb.at[pl.ds(0, tail)],
            acc.at[pl.ds(row0 + nrows - tail, tail)],
        )


def _zero_shared_rows(zb, acc, s):
    @pl.when(s < NS - 1)
    def _():
        _zero_rows(zb, acc, s * SUB_ROWS, SUB_ROWS)

    @pl.when(s == NS - 1)
    def _():
        _zero_rows(zb, acc, (NS - 1) * SUB_ROWS, SUB_ROWS_LAST)


def _copy_out_rows(acc, out_hbm, c, s):
    @pl.when(s < NS - 1)
    def _():
        row0 = s * SUB_ROWS
        pltpu.sync_copy(
            acc.at[pl.ds(row0, SUB_ROWS)],
            out_hbm.at[c].at[pl.ds(row0, SUB_ROWS)],
        )

    @pl.when(s == NS - 1)
    def _():
        row0 = (NS - 1) * SUB_ROWS
        pltpu.sync_copy(
            acc.at[pl.ds(row0, SUB_ROWS_LAST)],
            out_hbm.at[c].at[pl.ds(row0, SUB_ROWS_LAST)],
        )


@jax.jit
def _sc_degree(sd):
    @functools.partial(
        pl.kernel,
        mesh=_MESH,
        out_type=jax.ShapeDtypeStruct((NC, NPAD, D), jnp.float32),
        scratch_types=(
            [pltpu.VMEM((2, B), jnp.int32) for _ in range(DEPTH)]
            + [
                pltpu.VMEM((B, D), jnp.float32),
                pltpu.VMEM((128, D), jnp.float32),
                pltpu.VMEM_SHARED((NPAD, D), jnp.float32),
                pltpu.SemaphoreType.DMA,
            ]
        ),
    )
    def k(sd_hbm, out_hbm, idx0, idx1, idx2, ones_v, zb, acc, sem):
        idx = [idx0, idx1, idx2]
        c = lax.axis_index("c")
        s = lax.axis_index("s")
        wid = s * NC + c
        blk0 = wid * NBLK

        one = jnp.full((16,), 1.0, jnp.float32)

        @pl.loop(0, B)
        def _(r):
            @pl.loop(0, D, step=16)
            def _(col):
                ones_v[r, pl.ds(col, 16)] = one

        _zero_tile(zb, D)
        _zero_shared_rows(zb, acc, s)
        plsc.subcore_barrier()

        @pl.loop(0, NBLK, step=DEPTH)
        def _(i):
            idx_cps = [
                pltpu.async_copy(sd_hbm.at[blk0 + i + j], idx[j], sem)
                for j in range(DEPTH)
            ]
            for cp in idx_cps:
                cp.wait()
            for j in range(DEPTH):
                pltpu.sync_copy(ones_v, acc.at[idx[j].at[1]], add=True)

        plsc.subcore_barrier()
        _copy_out_rows(acc, out_hbm, c, s)

    return k(sd)


@jax.jit
def _sc_edge_pass(table, sd):
    @functools.partial(
        pl.kernel,
        mesh=_MESH,
        out_type=jax.ShapeDtypeStruct((NC, NPAD, D), jnp.float32),
        scratch_types=(
            [pltpu.VMEM((2, B), jnp.int32) for _ in range(6)]
            + [pltpu.VMEM((B, D), jnp.float32) for _ in range(DEPTH)]
            + [
                pltpu.VMEM_SHARED((NPAD, D), jnp.float32),
                pltpu.SemaphoreType.DMA,
            ]
            + [pltpu.SemaphoreType.DMA for _ in range(DEPTH)]
        ),
    )
    def k(tab_hbm, sd_hbm, out_hbm,
          ia0, ia1, ia2, ib0, ib1, ib2, rows0, rows1, rows2, acc,
          sem_i, sem_g0, sem_g1, sem_g2):
        c = lax.axis_index("c")
        s = lax.axis_index("s")
        wid = s * NC + c
        blk0 = wid * NBLK
        seta = [ia0, ia1, ia2]
        setb = [ib0, ib1, ib2]
        rows = [rows0, rows1, rows2]
        sem_g = [sem_g0, sem_g1, sem_g2]

        _zero_tile(rows0, D)
        _zero_shared_rows(rows0, acc, s)
        plsc.subcore_barrier()

        @pl.loop(0, NBLK, step=3 * DEPTH)
        def _(i):
            cps = [
                pltpu.async_copy(sd_hbm.at[blk0 + i + j], seta[j], sem_i)
                for j in range(DEPTH)
            ]
            for cp in cps:
                cp.wait()
            g_cps = [
                pltpu.async_copy(tab_hbm.at[seta[j].at[0]], rows[j], sem_g[j])
                for j in range(DEPTH)
            ]
            for grp, cur, nxt in ((1, seta, setb), (2, setb, seta)):
                idx_cps = [
                    pltpu.async_copy(
                        sd_hbm.at[blk0 + i + grp * DEPTH + j], nxt[j], sem_i)
                    for j in range(DEPTH)
                ]
                nxt_cps = []
                for j in range(DEPTH):
                    g_cps[j].wait()
                    pltpu.sync_copy(rows[j], acc.at[cur[j].at[1]], add=True)
                    if j == 0:
                        for cp in idx_cps:
                            cp.wait()
                    nxt_cps.append(pltpu.async_copy(
                        tab_hbm.at[nxt[j].at[0]], rows[j], sem_g[j]))
                g_cps = nxt_cps
            for j in range(DEPTH):
                g_cps[j].wait()
                pltpu.sync_copy(rows[j], acc.at[seta[j].at[1]], add=True)

        plsc.subcore_barrier()
        _copy_out_rows(acc, out_hbm, c, s)

    return k(table, sd)


def _dinv(deg_block):
    deg = deg_block[0, :, 0:1] + deg_block[1, :, 0:1]
    return jnp.where(deg > 0, lax.rsqrt(jnp.maximum(deg, 1e-12)), 0.0)


def _mma_body(x_ref, w_ref, o_ref):
    o_ref[...] = jnp.dot(x_ref[...], w_ref[...],
                         preferred_element_type=jnp.float32,
                         precision=lax.Precision.HIGHEST)


def _scale_body(h_ref, d_ref, o_ref):
    o_ref[...] = h_ref[...] * _dinv(d_ref[...])


def _mid_body(p_ref, d_ref, b_ref, w_ref, o_ref):
    dinv = _dinv(d_ref[...])
    y = jnp.maximum((p_ref[0] + p_ref[1]) * dinv + b_ref[...], 0.0)
    o_ref[...] = jnp.dot(y, w_ref[...], preferred_element_type=jnp.float32,
                         precision=lax.Precision.HIGHEST) * dinv


def _final_body(q_ref, d_ref, b_ref, o_ref):
    o_ref[...] = (q_ref[0] + q_ref[1]) * _dinv(d_ref[...]) + b_ref[...]


@jax.jit
def _tc_mma(Xp, W1):
    R, G = 2504, 4
    return pl.pallas_call(
        _mma_body,
        grid=(G,),
        in_specs=[
            pl.BlockSpec((R, D), lambda i: (i, 0)),
            pl.BlockSpec((D, D), lambda i: (0, 0)),
        ],
        out_specs=pl.BlockSpec((R, D), lambda i: (i, 0)),
        out_shape=jax.ShapeDtypeStruct((NPAD, D), jnp.float32),
    )(Xp, W1)


@jax.jit
def _tc_scale(h, degp):
    R, G = 2504, 4
    return pl.pallas_call(
        _scale_body,
        grid=(G,),
        in_specs=[
            pl.BlockSpec((R, D), lambda i: (i, 0)),
            pl.BlockSpec((NC, R, D), lambda i: (0, i, 0)),
        ],
        out_specs=pl.BlockSpec((R, D), lambda i: (i, 0)),
        out_shape=jax.ShapeDtypeStruct((NPAD, D), jnp.float32),
    )(h, degp)


@jax.jit
def _tc_mid(p, degp, b1, W2):
    R, G = 2504, 4
    return pl.pallas_call(
        _mid_body,
        grid=(G,),
        in_specs=[
            pl.BlockSpec((NC, R, D), lambda i: (0, i, 0)),
            pl.BlockSpec((NC, R, D), lambda i: (0, i, 0)),
            pl.BlockSpec((1, D), lambda i: (0, 0)),
            pl.BlockSpec((D, D), lambda i: (0, 0)),
        ],
        out_specs=pl.BlockSpec((R, D), lambda i: (i, 0)),
        out_shape=jax.ShapeDtypeStruct((NPAD, D), jnp.float32),
    )(p, degp, b1.reshape(1, D), W2)


@jax.jit
def _tc_final(q, degp, b2):
    R, G = 2000, 5
    return pl.pallas_call(
        _final_body,
        grid=(G,),
        in_specs=[
            pl.BlockSpec((NC, R, D), lambda i: (0, i, 0)),
            pl.BlockSpec((NC, R, D), lambda i: (0, i, 0)),
            pl.BlockSpec((1, D), lambda i: (0, 0)),
        ],
        out_specs=pl.BlockSpec((R, D), lambda i: (i, 0)),
        out_shape=jax.ShapeDtypeStruct((N, D), jnp.float32),
    )(q, degp, b2.reshape(1, D))


@jax.jit
def kernel(X, A, W1, b1, W2, b2):
    pad = jnp.asarray(jnp.arange(EPAD - E) % 16 + N, jnp.int32)
    srcp = jnp.concatenate([A[0].astype(jnp.int32), pad])
    dstp = jnp.concatenate([A[1].astype(jnp.int32), pad])
    sd = jnp.stack([srcp.reshape(EPAD // B, B), dstp.reshape(EPAD // B, B)],
                   axis=1)
    Xp = jnp.zeros((NPAD, D), jnp.float32).at[:N].set(X)

    h = _tc_mma(Xp, W1)
    degp = _sc_degree(sd)
    hs = _tc_scale(h, degp)
    p = _sc_edge_pass(hs, sd)
    hs2 = _tc_mid(p, degp, b1, W2)
    q = _sc_edge_pass(hs2, sd)
    return _tc_final(q, degp, b2)


# speedup vs baseline: 21.6881x; 1.0194x over previous
import functools

import jax
import jax.numpy as jnp
from jax import lax
from jax.experimental import pallas as pl
from jax.experimental.pallas import tpu as pltpu
from jax.experimental.pallas import tpu_sc as plsc

N = 10000
NPAD = 10016
E = 320000
D = 128

NC, NS = 2, 16
NW = NC * NS
B = 128
DEPTH = 3
NBLK = 81
EPAD = NBLK * B * NW
SUB_ROWS = 632
SUB_ROWS_LAST = NPAD - 15 * SUB_ROWS

_MESH = plsc.VectorSubcoreMesh(core_axis_name="c", subcore_axis_name="s")


def _zero_tile(zb, width):
    zero = jnp.zeros((16,), jnp.float32)

    @pl.loop(0, 128)
    def _(r):
        @pl.loop(0, width, step=16)
        def _(col):
            zb[r, pl.ds(col, 16)] = zero


def _zero_rows(zb, acc, row0, nrows):
    tail = nrows % 128

    @pl.loop(0, nrows - tail, step=128)
    def _(r0):
        pltpu.sync_copy(zb, acc.at[pl.ds(row0 + r0, 128)])

    if tail:
        pltpu.sync_copy(
            zb.at[pl.ds(0, tail)],
            acc.at[pl.ds(row0 + nrows - tail, tail)],
        )


def _zero_shared_rows(zb, acc, s):
    @pl.when(s < NS - 1)
    def _():
        _zero_rows(zb, acc, s * SUB_ROWS, SUB_ROWS)

    @pl.when(s == NS - 1)
    def _():
        _zero_rows(zb, acc, (NS - 1) * SUB_ROWS, SUB_ROWS_LAST)


def _copy_out_rows(acc, out_hbm, c, s):
    @pl.when(s < NS - 1)
    def _():
        row0 = s * SUB_ROWS
        pltpu.sync_copy(
            acc.at[pl.ds(row0, SUB_ROWS)],
            out_hbm.at[c].at[pl.ds(row0, SUB_ROWS)],
        )

    @pl.when(s == NS - 1)
    def _():
        row0 = (NS - 1) * SUB_ROWS
        pltpu.sync_copy(
            acc.at[pl.ds(row0, SUB_ROWS_LAST)],
            out_hbm.at[c].at[pl.ds(row0, SUB_ROWS_LAST)],
        )


@jax.jit
def _sc_degree(sd):
    @functools.partial(
        pl.kernel,
        mesh=_MESH,
        out_type=jax.ShapeDtypeStruct((NC, NPAD, D), jnp.float32),
        scratch_types=(
            [pltpu.VMEM((2, B), jnp.int32) for _ in range(6)]
            + [
                pltpu.VMEM((B, D), jnp.float32),
                pltpu.VMEM((128, D), jnp.float32),
                pltpu.VMEM_SHARED((NPAD, D), jnp.float32),
                pltpu.SemaphoreType.DMA,
            ]
        ),
    )
    def k(sd_hbm, out_hbm, ia0, ia1, ia2, ib0, ib1, ib2, ones_v, zb, acc, sem):
        seta = [ia0, ia1, ia2]
        setb = [ib0, ib1, ib2]
        c = lax.axis_index("c")
        s = lax.axis_index("s")
        wid = s * NC + c
        blk0 = wid * NBLK

        one = jnp.full((16,), 1.0, jnp.float32)

        @pl.loop(0, B)
        def _(r):
            @pl.loop(0, D, step=16)
            def _(col):
                ones_v[r, pl.ds(col, 16)] = one

        _zero_tile(zb, D)
        _zero_shared_rows(zb, acc, s)
        plsc.subcore_barrier()

        @pl.loop(0, NBLK, step=3 * DEPTH)
        def _(i):
            cps = [
                pltpu.async_copy(sd_hbm.at[blk0 + i + j], seta[j], sem)
                for j in range(DEPTH)
            ]
            for cp in cps:
                cp.wait()
            for grp, cur, nxt in ((1, seta, setb), (2, setb, seta)):
                idx_cps = [
                    pltpu.async_copy(
                        sd_hbm.at[blk0 + i + grp * DEPTH + j], nxt[j], sem)
                    for j in range(DEPTH)
                ]
                for j in range(DEPTH):
                    pltpu.sync_copy(ones_v, acc.at[cur[j].at[1]], add=True)
                for cp in idx_cps:
                    cp.wait()
            for j in range(DEPTH):
                pltpu.sync_copy(ones_v, acc.at[seta[j].at[1]], add=True)

        plsc.subcore_barrier()
        _copy_out_rows(acc, out_hbm, c, s)

    return k(sd)


@jax.jit
def _sc_edge_pass(table, sd):
    @functools.partial(
        pl.kernel,
        mesh=_MESH,
        out_type=jax.ShapeDtypeStruct((NC, NPAD, D), jnp.float32),
        scratch_types=(
            [pltpu.VMEM((2, B), jnp.int32) for _ in range(6)]
            + [pltpu.VMEM((B, D), jnp.float32) for _ in range(DEPTH)]
            + [
                pltpu.VMEM_SHARED((NPAD, D), jnp.float32),
                pltpu.SemaphoreType.DMA,
            ]
            + [pltpu.SemaphoreType.DMA for _ in range(DEPTH)]
        ),
    )
    def k(tab_hbm, sd_hbm, out_hbm,
          ia0, ia1, ia2, ib0, ib1, ib2, rows0, rows1, rows2, acc,
          sem_i, sem_g0, sem_g1, sem_g2):
        c = lax.axis_index("c")
        s = lax.axis_index("s")
        wid = s * NC + c
        blk0 = wid * NBLK
        seta = [ia0, ia1, ia2]
        setb = [ib0, ib1, ib2]
        rows = [rows0, rows1, rows2]
        sem_g = [sem_g0, sem_g1, sem_g2]

        _zero_tile(rows0, D)
        _zero_shared_rows(rows0, acc, s)
        plsc.subcore_barrier()

        @pl.loop(0, NBLK, step=3 * DEPTH)
        def _(i):
            cps = [
                pltpu.async_copy(sd_hbm.at[blk0 + i + j], seta[j], sem_i)
                for j in range(DEPTH)
            ]
            for cp in cps:
                cp.wait()
            g_cps = [
                pltpu.async_copy(tab_hbm.at[seta[j].at[0]], rows[j], sem_g[j])
                for j in range(DEPTH)
            ]
            for grp, cur, nxt in ((1, seta, setb), (2, setb, seta)):
                idx_cps = [
                    pltpu.async_copy(
                        sd_hbm.at[blk0 + i + grp * DEPTH + j], nxt[j], sem_i)
                    for j in range(DEPTH)
                ]
                nxt_cps = []
                for j in range(DEPTH):
                    g_cps[j].wait()
                    pltpu.sync_copy(rows[j], acc.at[cur[j].at[1]], add=True)
                    if j == 0:
                        for cp in idx_cps:
                            cp.wait()
                    nxt_cps.append(pltpu.async_copy(
                        tab_hbm.at[nxt[j].at[0]], rows[j], sem_g[j]))
                g_cps = nxt_cps
            for j in range(DEPTH):
                g_cps[j].wait()
                pltpu.sync_copy(rows[j], acc.at[seta[j].at[1]], add=True)

        plsc.subcore_barrier()
        _copy_out_rows(acc, out_hbm, c, s)

    return k(table, sd)


def _dinv(deg_block):
    deg = deg_block[0, :, 0:1] + deg_block[1, :, 0:1]
    return jnp.where(deg > 0, lax.rsqrt(jnp.maximum(deg, 1e-12)), 0.0)


def _mma_body(x_ref, w_ref, o_ref):
    o_ref[...] = jnp.dot(x_ref[...], w_ref[...],
                         preferred_element_type=jnp.float32,
                         precision=lax.Precision.HIGHEST)


def _scale_body(h_ref, d_ref, o_ref):
    o_ref[...] = h_ref[...] * _dinv(d_ref[...])


def _mid_body(p_ref, d_ref, b_ref, w_ref, o_ref):
    dinv = _dinv(d_ref[...])
    y = jnp.maximum((p_ref[0] + p_ref[1]) * dinv + b_ref[...], 0.0)
    o_ref[...] = jnp.dot(y, w_ref[...], preferred_element_type=jnp.float32,
                         precision=lax.Precision.HIGHEST) * dinv


def _final_body(q_ref, d_ref, b_ref, o_ref):
    o_ref[...] = (q_ref[0] + q_ref[1]) * _dinv(d_ref[...]) + b_ref[...]


@jax.jit
def _tc_mma(Xp, W1):
    R, G = 2504, 4
    return pl.pallas_call(
        _mma_body,
        grid=(G,),
        in_specs=[
            pl.BlockSpec((R, D), lambda i: (i, 0)),
            pl.BlockSpec((D, D), lambda i: (0, 0)),
        ],
        out_specs=pl.BlockSpec((R, D), lambda i: (i, 0)),
        out_shape=jax.ShapeDtypeStruct((NPAD, D), jnp.float32),
    )(Xp, W1)


@jax.jit
def _tc_scale(h, degp):
    R, G = 2504, 4
    return pl.pallas_call(
        _scale_body,
        grid=(G,),
        in_specs=[
            pl.BlockSpec((R, D), lambda i: (i, 0)),
            pl.BlockSpec((NC, R, D), lambda i: (0, i, 0)),
        ],
        out_specs=pl.BlockSpec((R, D), lambda i: (i, 0)),
        out_shape=jax.ShapeDtypeStruct((NPAD, D), jnp.float32),
    )(h, degp)


@jax.jit
def _tc_mid(p, degp, b1, W2):
    R, G = 2504, 4
    return pl.pallas_call(
        _mid_body,
        grid=(G,),
        in_specs=[
            pl.BlockSpec((NC, R, D), lambda i: (0, i, 0)),
            pl.BlockSpec((NC, R, D), lambda i: (0, i, 0)),
            pl.BlockSpec((1, D), lambda i: (0, 0)),
            pl.BlockSpec((D, D), lambda i: (0, 0)),
        ],
        out_specs=pl.BlockSpec((R, D), lambda i: (i, 0)),
        out_shape=jax.ShapeDtypeStruct((NPAD, D), jnp.float32),
    )(p, degp, b1.reshape(1, D), W2)


@jax.jit
def _tc_final(q, degp, b2):
    R, G = 2000, 5
    return pl.pallas_call(
        _final_body,
        grid=(G,),
        in_specs=[
            pl.BlockSpec((NC, R, D), lambda i: (0, i, 0)),
            pl.BlockSpec((NC, R, D), lambda i: (0, i, 0)),
            pl.BlockSpec((1, D), lambda i: (0, 0)),
        ],
        out_specs=pl.BlockSpec((R, D), lambda i: (i, 0)),
        out_shape=jax.ShapeDtypeStruct((N, D), jnp.float32),
    )(q, degp, b2.reshape(1, D))


@jax.jit
def kernel(X, A, W1, b1, W2, b2):
    pad = jnp.asarray(jnp.arange(EPAD - E) % 16 + N, jnp.int32)
    srcp = jnp.concatenate([A[0].astype(jnp.int32), pad])
    dstp = jnp.concatenate([A[1].astype(jnp.int32), pad])
    sd = jnp.stack([srcp.reshape(EPAD // B, B), dstp.reshape(EPAD // B, B)],
                   axis=1)
    Xp = jnp.zeros((NPAD, D), jnp.float32).at[:N].set(X)

    h = _tc_mma(Xp, W1)
    degp = _sc_degree(sd)
    hs = _tc_scale(h, degp)
    p = _sc_edge_pass(hs, sd)
    hs2 = _tc_mid(p, degp, b1, W2)
    q = _sc_edge_pass(hs2, sd)
    return _tc_final(q, degp, b2)
